# baseline (speedup 1.0000x reference)
import functools

import jax
import jax.numpy as jnp
from jax.experimental import pallas as pl
from jax.experimental.pallas import tpu as pltpu


def _moe_dense_kernel(x_ref, xe_ref, gw_ref, gb_ref, w_ref, b_ref, out_ref,
                      p_ref, *, bt, n_experts):
    n = pl.program_id(1)
    e = pl.program_id(2)

    @pl.when(jnp.logical_and(n == 0, e == 0))
    def _compute_gate():
        logits = jnp.dot(x_ref[...], gw_ref[...],
                         preferred_element_type=jnp.float32) + gb_ref[...]
        iota = jax.lax.broadcasted_iota(jnp.int32, (bt, n_experts), 1)
        m1 = jnp.max(logits, axis=1, keepdims=True)
        i1 = jnp.min(jnp.where(logits == m1, iota, n_experts), axis=1,
                     keepdims=True)
        f1 = iota == i1
        l2 = jnp.where(f1, -jnp.inf, logits)
        m2 = jnp.max(l2, axis=1, keepdims=True)
        i2 = jnp.min(jnp.where(l2 == m2, iota, n_experts), axis=1,
                     keepdims=True)
        f2 = iota == i2
        p1 = 1.0 / (1.0 + jnp.exp(m2 - m1))
        p2 = 1.0 - p1
        p_ref[...] = (p1 * f1.astype(jnp.float32)
                      + p2 * f2.astype(jnp.float32))

    probs = p_ref[...]
    eidx = jax.lax.broadcasted_iota(jnp.int32, (bt, n_experts), 1)
    w_tok = jnp.sum(probs * (eidx == e).astype(jnp.float32), axis=1,
                    keepdims=True)
    contrib = w_tok * jnp.dot(xe_ref[...], w_ref[0],
                              preferred_element_type=jnp.float32)

    @pl.when(e == 0)
    def _init():
        out_ref[...] = jnp.dot(probs, b_ref[...],
                               preferred_element_type=jnp.float32) + contrib

    @pl.when(e != 0)
    def _acc():
        out_ref[...] += contrib


def kernel(insample_y, gate_W, gate_b, expert_W, expert_b):
    n_tok, d_model = insample_y.shape
    n_experts, _, out_features = expert_W.shape
    bt = min(512, n_tok)
    bn = min(512, out_features)
    grid = (n_tok // bt, out_features // bn, n_experts)

    x = insample_y
    xe = insample_y
    gb = gate_b.reshape(1, n_experts)

    fn = functools.partial(_moe_dense_kernel, bt=bt, n_experts=n_experts)
    return pl.pallas_call(
        fn,
        grid=grid,
        in_specs=[
            pl.BlockSpec((bt, d_model), lambda t, n, e: (t, 0)),
            pl.BlockSpec((bt, d_model), lambda t, n, e: (t, 0)),
            pl.BlockSpec((d_model, n_experts), lambda t, n, e: (0, 0)),
            pl.BlockSpec((1, n_experts), lambda t, n, e: (0, 0)),
            pl.BlockSpec((1, d_model, bn), lambda t, n, e: (e, 0, n)),
            pl.BlockSpec((n_experts, bn), lambda t, n, e: (0, n)),
        ],
        out_specs=pl.BlockSpec((bt, bn), lambda t, n, e: (t, n)),
        out_shape=jax.ShapeDtypeStruct((n_tok, out_features), jnp.float32),
        scratch_shapes=[pltpu.VMEM((bt, n_experts), jnp.float32)],
        compiler_params=pltpu.CompilerParams(
            dimension_semantics=("parallel", "parallel", "arbitrary")),
    )(x, xe, gate_W, gb, expert_W, expert_b)


# speedup vs baseline: 1.0701x; 1.0701x over previous
import functools

import jax
import jax.numpy as jnp
from jax.experimental import pallas as pl
from jax.experimental.pallas import tpu as pltpu


def _moe_dense_kernel(x_ref, xe_ref, gw_ref, gb_ref, w_ref, b_ref, out_ref,
                      p_ref, *, bt, n_experts):
    n = pl.program_id(1)
    e = pl.program_id(2)

    @pl.when(jnp.logical_and(n == 0, e == 0))
    def _compute_gate():
        logits = jnp.dot(x_ref[...], gw_ref[...],
                         preferred_element_type=jnp.float32) + gb_ref[...]
        iota = jax.lax.broadcasted_iota(jnp.int32, (bt, n_experts), 1)
        m1 = jnp.max(logits, axis=1, keepdims=True)
        i1 = jnp.min(jnp.where(logits == m1, iota, n_experts), axis=1,
                     keepdims=True)
        f1 = iota == i1
        l2 = jnp.where(f1, -jnp.inf, logits)
        m2 = jnp.max(l2, axis=1, keepdims=True)
        i2 = jnp.min(jnp.where(l2 == m2, iota, n_experts), axis=1,
                     keepdims=True)
        f2 = iota == i2
        p1 = 1.0 / (1.0 + jnp.exp(m2 - m1))
        p2 = 1.0 - p1
        p_ref[...] = (p1 * f1.astype(jnp.float32)
                      + p2 * f2.astype(jnp.float32))

    probs = p_ref[...]
    eidx = jax.lax.broadcasted_iota(jnp.int32, (bt, n_experts), 1)
    w_tok = jnp.sum(probs * (eidx == e).astype(jnp.float32), axis=1,
                    keepdims=True)
    contrib = w_tok * jnp.dot(xe_ref[...], w_ref[0],
                              preferred_element_type=jnp.float32)

    @pl.when(e == 0)
    def _init():
        out_ref[...] = jnp.dot(probs, b_ref[...],
                               preferred_element_type=jnp.float32) + contrib

    @pl.when(e != 0)
    def _acc():
        out_ref[...] += contrib


def kernel(insample_y, gate_W, gate_b, expert_W, expert_b):
    n_tok, d_model = insample_y.shape
    n_experts, _, out_features = expert_W.shape
    bt = min(512, n_tok)
    bn = min(512, out_features)
    grid = (n_tok // bt, out_features // bn, n_experts)

    x = insample_y
    xe = insample_y.astype(jnp.bfloat16)
    ew = expert_W.astype(jnp.bfloat16)
    gb = gate_b.reshape(1, n_experts)

    fn = functools.partial(_moe_dense_kernel, bt=bt, n_experts=n_experts)
    return pl.pallas_call(
        fn,
        grid=grid,
        in_specs=[
            pl.BlockSpec((bt, d_model), lambda t, n, e: (t, 0)),
            pl.BlockSpec((bt, d_model), lambda t, n, e: (t, 0)),
            pl.BlockSpec((d_model, n_experts), lambda t, n, e: (0, 0)),
            pl.BlockSpec((1, n_experts), lambda t, n, e: (0, 0)),
            pl.BlockSpec((1, d_model, bn), lambda t, n, e: (e, 0, n)),
            pl.BlockSpec((n_experts, bn), lambda t, n, e: (0, n)),
        ],
        out_specs=pl.BlockSpec((bt, bn), lambda t, n, e: (t, n)),
        out_shape=jax.ShapeDtypeStruct((n_tok, out_features), jnp.float32),
        scratch_shapes=[pltpu.VMEM((bt, n_experts), jnp.float32)],
        compiler_params=pltpu.CompilerParams(
            dimension_semantics=("parallel", "parallel", "arbitrary")),
    )(x, xe, gate_W, gb, ew, expert_b)


# speedup vs baseline: 1.3339x; 1.2465x over previous
import functools

import jax
import jax.numpy as jnp
from jax.experimental import pallas as pl
from jax.experimental.pallas import tpu as pltpu


def _moe_dense_kernel(x_ref, xe_ref, gw_ref, gb_ref, w_ref, b_ref, out_ref,
                      p_ref, *, bt, n_experts):
    n = pl.program_id(1)
    e = pl.program_id(2)

    @pl.when(jnp.logical_and(n == 0, e == 0))
    def _compute_gate():
        logits = jnp.dot(x_ref[...], gw_ref[...],
                         preferred_element_type=jnp.float32) + gb_ref[...]
        iota = jax.lax.broadcasted_iota(jnp.int32, (bt, n_experts), 1)
        m1 = jnp.max(logits, axis=1, keepdims=True)
        i1 = jnp.min(jnp.where(logits == m1, iota, n_experts), axis=1,
                     keepdims=True)
        f1 = iota == i1
        l2 = jnp.where(f1, -jnp.inf, logits)
        m2 = jnp.max(l2, axis=1, keepdims=True)
        i2 = jnp.min(jnp.where(l2 == m2, iota, n_experts), axis=1,
                     keepdims=True)
        f2 = iota == i2
        p1 = 1.0 / (1.0 + jnp.exp(m2 - m1))
        p2 = 1.0 - p1
        p_ref[...] = (p1 * f1.astype(jnp.float32)
                      + p2 * f2.astype(jnp.float32))

    probs = p_ref[...]
    eidx = jax.lax.broadcasted_iota(jnp.int32, (bt, n_experts), 1)
    w_tok = jnp.sum(probs * (eidx == e).astype(jnp.float32), axis=1,
                    keepdims=True)
    contrib = w_tok * jnp.dot(xe_ref[...], w_ref[0],
                              preferred_element_type=jnp.float32)

    @pl.when(e == 0)
    def _init():
        out_ref[...] = jnp.dot(probs, b_ref[...],
                               preferred_element_type=jnp.float32) + contrib

    @pl.when(e != 0)
    def _acc():
        out_ref[...] += contrib


def kernel(insample_y, gate_W, gate_b, expert_W, expert_b):
    n_tok, d_model = insample_y.shape
    n_experts, _, out_features = expert_W.shape
    bt = min(1024, n_tok)
    bn = min(1024, out_features)
    grid = (n_tok // bt, out_features // bn, n_experts)

    x = insample_y
    xe = insample_y.astype(jnp.bfloat16)
    ew = expert_W.astype(jnp.bfloat16)
    gb = gate_b.reshape(1, n_experts)

    fn = functools.partial(_moe_dense_kernel, bt=bt, n_experts=n_experts)
    return pl.pallas_call(
        fn,
        grid=grid,
        in_specs=[
            pl.BlockSpec((bt, d_model), lambda t, n, e: (t, 0)),
            pl.BlockSpec((bt, d_model), lambda t, n, e: (t, 0)),
            pl.BlockSpec((d_model, n_experts), lambda t, n, e: (0, 0)),
            pl.BlockSpec((1, n_experts), lambda t, n, e: (0, 0)),
            pl.BlockSpec((1, d_model, bn), lambda t, n, e: (e, 0, n)),
            pl.BlockSpec((n_experts, bn), lambda t, n, e: (0, n)),
        ],
        out_specs=pl.BlockSpec((bt, bn), lambda t, n, e: (t, n)),
        out_shape=jax.ShapeDtypeStruct((n_tok, out_features), jnp.float32),
        scratch_shapes=[pltpu.VMEM((bt, n_experts), jnp.float32)],
        compiler_params=pltpu.CompilerParams(
            dimension_semantics=("parallel", "parallel", "arbitrary")),
    )(x, xe, gate_W, gb, ew, expert_b)
